# baseline (speedup 1.0000x reference)
import functools

import jax
import jax.numpy as jnp
from jax import lax
from jax.experimental import pallas as pl
from jax.experimental.pallas import tpu as pltpu
from jax.experimental.pallas import tpu_sc as plsc

N_NODES = 10000
N_PAD = 10240
N_EDGES = 320000
NW = 32
EPW = N_EDGES // NW
CH = 125
NCH = EPW // CH
RPT = N_PAD // 16
ZCH = 128
NBUF = 5


@functools.lru_cache(maxsize=None)
def _sc_agg(D):
    mesh = plsc.VectorSubcoreMesh(core_axis_name="c", subcore_axis_name="s")

    @functools.partial(
        pl.kernel,
        mesh=mesh,
        out_type=[
            jax.ShapeDtypeStruct((N_PAD, 128), jnp.float32),
            jax.ShapeDtypeStruct((N_PAD, 128), jnp.float32),
        ],
        scratch_types=[
            pltpu.VMEM((NCH, CH), jnp.int32),
            pltpu.VMEM((NCH, CH), jnp.int32),
            [pltpu.VMEM((CH, D), jnp.float32) for _ in range(NBUF)],
            pltpu.VMEM((ZCH, D), jnp.float32),
            pltpu.VMEM_SHARED((N_PAD, D), jnp.float32),
            [pltpu.SemaphoreType.DMA for _ in range(NBUF)],
            [pltpu.SemaphoreType.DMA for _ in range(NBUF)],
            pltpu.SemaphoreType.DMA,
        ],
        compiler_params=pltpu.CompilerParams(use_tc_tiling_on_sc=False),
    )
    def k(table, src, dst, out0, out1, src_v, dst_v, bufs, zbuf, acc, gs, ss,
          zs):
        cid = lax.axis_index("c")
        sid = lax.axis_index("s")
        wid = sid * 2 + cid
        cps = pltpu.async_copy(src.at[pl.ds(wid * NCH, NCH)], src_v, gs[0])
        cpd = pltpu.async_copy(dst.at[pl.ds(wid * NCH, NCH)], dst_v, gs[1])

        zeros = jnp.zeros((16,), jnp.float32)

        def zrow(i, _):
            for d in range(D // 16):
                zbuf[i, pl.ds(d * 16, 16)] = zeros
            return 0

        lax.fori_loop(0, ZCH, zrow, 0)
        row0 = sid * RPT
        for kk in range(RPT // ZCH):
            pltpu.async_copy(zbuf, acc.at[pl.ds(row0 + kk * ZCH, ZCH)], zs)
        for kk in range(RPT // ZCH):
            pltpu.make_async_copy(
                zbuf, acc.at[pl.ds(row0 + kk * ZCH, ZCH)], zs).wait()
        cps.wait()
        cpd.wait()
        plsc.subcore_barrier()

        def gather(j, b):
            return pltpu.async_copy(table.at[src_v.at[j]], bufs[b], gs[b])

        def scat(j, b):
            return pltpu.async_copy(bufs[b], acc.at[dst_v.at[j]], ss[b],
                                    add=True)

        for b in range(NBUF - 1):
            gather(b, b)

        def body(g, _):
            j0 = NBUF * g
            for b in range(NBUF):
                j = j0 + b
                pltpu.make_async_copy(table.at[src_v.at[j]], bufs[b],
                                      gs[b]).wait()
                scat(j, b)
                bn = (b + NBUF - 1) % NBUF

                @pl.when(j + NBUF - 1 < NCH)
                def _():
                    @pl.when(j >= 1)
                    def _():
                        pltpu.make_async_copy(
                            bufs[bn], acc.at[dst_v.at[j - 1]], ss[bn]).wait()

                    gather(j + NBUF - 1, bn)
            return 0

        lax.fori_loop(0, NCH // NBUF, body, 0)
        for jj in range(NCH - NBUF, NCH):
            pltpu.make_async_copy(
                bufs[jj % NBUF], acc.at[dst_v.at[jj]], ss[jj % NBUF]).wait()
        plsc.subcore_barrier()

        @pl.when(cid == 0)
        def _():
            pltpu.sync_copy(acc.at[pl.ds(row0, RPT)],
                            out0.at[pl.ds(row0, RPT), pl.ds(0, D)])

        @pl.when(cid == 1)
        def _():
            pltpu.sync_copy(acc.at[pl.ds(row0, RPT)],
                            out1.at[pl.ds(row0, RPT), pl.ds(0, D)])

    return k


@functools.lru_cache(maxsize=None)
def _make_sc_deg():
    mesh = plsc.VectorSubcoreMesh(core_axis_name="c", subcore_axis_name="s")
    D = 16

    @functools.partial(
        pl.kernel,
        mesh=mesh,
        out_type=[
            jax.ShapeDtypeStruct((N_PAD, 128), jnp.float32),
            jax.ShapeDtypeStruct((N_PAD, 128), jnp.float32),
        ],
        scratch_types=[
            pltpu.VMEM((NCH, CH), jnp.int32),
            pltpu.VMEM((CH, D), jnp.float32),
            pltpu.VMEM((ZCH, D), jnp.float32),
            pltpu.VMEM_SHARED((N_PAD, D), jnp.float32),
            [pltpu.SemaphoreType.DMA for _ in range(4)],
            pltpu.SemaphoreType.DMA,
            pltpu.SemaphoreType.DMA,
        ],
        compiler_params=pltpu.CompilerParams(use_tc_tiling_on_sc=False),
    )
    def k(dst, out0, out1, dst_v, buf, zbuf, acc, ss, s0, zs):
        cid = lax.axis_index("c")
        sid = lax.axis_index("s")
        wid = sid * 2 + cid
        cpd = pltpu.async_copy(dst.at[pl.ds(wid * NCH, NCH)], dst_v, zs)

        zeros = jnp.zeros((16,), jnp.float32)

        def zrow(i, _):
            zbuf[i, :] = zeros
            return 0

        lax.fori_loop(0, ZCH, zrow, 0)
        row0 = sid * RPT
        for kk in range(RPT // ZCH):
            pltpu.async_copy(zbuf, acc.at[pl.ds(row0 + kk * ZCH, ZCH)], s0)
        ones = jnp.ones((16,), jnp.float32)

        def orow(i, _):
            buf[i, :] = ones
            return 0

        lax.fori_loop(0, CH, orow, 0)
        for kk in range(RPT // ZCH):
            pltpu.make_async_copy(
                zbuf, acc.at[pl.ds(row0 + kk * ZCH, ZCH)], s0).wait()
        cpd.wait()
        plsc.subcore_barrier()

        def body(g, _):
            j0 = 4 * g
            for b in range(4):
                j = j0 + b

                @pl.when(j >= 4)
                def _():
                    pltpu.make_async_copy(
                        buf, acc.at[dst_v.at[j - 4]], ss[b]).wait()

                pltpu.async_copy(buf, acc.at[dst_v.at[j]], ss[b], add=True)
            return 0

        lax.fori_loop(0, NCH // 4, body, 0)
        for jj in range(NCH - 4, NCH):
            pltpu.make_async_copy(buf, acc.at[dst_v.at[jj]], ss[jj % 4]).wait()
        plsc.subcore_barrier()

        @pl.when(cid == 0)
        def _():
            pltpu.sync_copy(acc.at[pl.ds(row0, RPT)],
                            out0.at[pl.ds(row0, RPT), pl.ds(0, D)])

        @pl.when(cid == 1)
        def _():
            pltpu.sync_copy(acc.at[pl.ds(row0, RPT)],
                            out1.at[pl.ds(row0, RPT), pl.ds(0, D)])

    return k


_R = 2000


def _dinv_of(d0_ref, d1_ref):
    deg = d0_ref[:, 0:1] + d1_ref[:, 0:1] + 1.0
    return lax.rsqrt(deg)


def _mm1_body(x_ref, w_ref, o_ref):
    o_ref[...] = jnp.dot(x_ref[...], w_ref[...],
                         preferred_element_type=jnp.float32)


def _tc_mm1(x, W1):
    return pl.pallas_call(
        _mm1_body,
        grid=(N_NODES // _R,),
        in_specs=[
            pl.BlockSpec((_R, 128), lambda i: (i, 0)),
            pl.BlockSpec((128, 64), lambda i: (0, 0)),
        ],
        out_specs=pl.BlockSpec((_R, 64), lambda i: (i, 0)),
        out_shape=jax.ShapeDtypeStruct((N_NODES, 64), jnp.float32),
    )(x, W1)


def _scale_body(h_ref, d0_ref, d1_ref, o_ref):
    o_ref[...] = h_ref[...] * _dinv_of(d0_ref, d1_ref)


def _tc_scale(h, dg0, dg1):
    return pl.pallas_call(
        _scale_body,
        grid=(N_NODES // _R,),
        in_specs=[
            pl.BlockSpec((_R, 64), lambda i: (i, 0)),
            pl.BlockSpec((_R, 128), lambda i: (i, 0)),
            pl.BlockSpec((_R, 128), lambda i: (i, 0)),
        ],
        out_specs=pl.BlockSpec((_R, 64), lambda i: (i, 0)),
        out_shape=jax.ShapeDtypeStruct((N_NODES, 64), jnp.float32),
    )(h, dg0, dg1)


def _mid_body(p0_ref, p1_ref, hp_ref, d0_ref, d1_ref, b1_ref, w2_ref, o_ref):
    dinv = _dinv_of(d0_ref, d1_ref)
    agg = p0_ref[:, :64] + p1_ref[:, :64]
    t = b1_ref[...] + dinv * (agg + hp_ref[...])
    t = jnp.maximum(t, 0.0)
    w2p = jnp.pad(w2_ref[...], ((0, 0), (0, 8)))
    o_ref[...] = jnp.dot(t, w2p, preferred_element_type=jnp.float32) * dinv


def _tc_mid(p0, p1, hp, dg0, dg1, b1r, W2r):
    return pl.pallas_call(
        _mid_body,
        grid=(N_NODES // _R,),
        in_specs=[
            pl.BlockSpec((_R, 128), lambda i: (i, 0)),
            pl.BlockSpec((_R, 128), lambda i: (i, 0)),
            pl.BlockSpec((_R, 64), lambda i: (i, 0)),
            pl.BlockSpec((_R, 128), lambda i: (i, 0)),
            pl.BlockSpec((_R, 128), lambda i: (i, 0)),
            pl.BlockSpec((1, 64), lambda i: (0, 0)),
            pl.BlockSpec((64, 40), lambda i: (0, 0)),
        ],
        out_specs=pl.BlockSpec((_R, 48), lambda i: (i, 0)),
        out_shape=jax.ShapeDtypeStruct((N_NODES, 48), jnp.float32),
    )(p0, p1, hp, dg0, dg1, b1r, W2r)


def _fin_body(p0_ref, p1_ref, hp_ref, d0_ref, d1_ref, b2_ref, o_ref):
    dinv = _dinv_of(d0_ref, d1_ref)
    b2p = jnp.pad(b2_ref[...], ((0, 0), (0, 8)))
    agg = p0_ref[:, :48] + p1_ref[:, :48]
    z = b2p + dinv * (agg + hp_ref[...])
    z = jnp.maximum(z, 0.0)
    col = lax.broadcasted_iota(jnp.int32, z.shape, 1)
    valid = col < 40
    zm = jnp.where(valid, z, -jnp.inf)
    m = jnp.max(zm, axis=1, keepdims=True)
    e = jnp.where(valid, jnp.exp(z - m), 0.0)
    s = jnp.log(jnp.sum(e, axis=1, keepdims=True))
    o_ref[...] = (z - m - s)[:, :40]


def _tc_fin(p0, p1, hp, dg0, dg1, b2r):
    return pl.pallas_call(
        _fin_body,
        grid=(N_NODES // _R,),
        in_specs=[
            pl.BlockSpec((_R, 128), lambda i: (i, 0)),
            pl.BlockSpec((_R, 128), lambda i: (i, 0)),
            pl.BlockSpec((_R, 48), lambda i: (i, 0)),
            pl.BlockSpec((_R, 128), lambda i: (i, 0)),
            pl.BlockSpec((_R, 128), lambda i: (i, 0)),
            pl.BlockSpec((1, 40), lambda i: (0, 0)),
        ],
        out_specs=pl.BlockSpec((_R, 40), lambda i: (i, 0)),
        out_shape=jax.ShapeDtypeStruct((N_NODES, 40), jnp.float32),
    )(p0, p1, hp, dg0, dg1, b2r)


def kernel(x, W1, b1, W2, b2, edge_index):
    ei = edge_index.astype(jnp.int32)
    src2 = ei[0].reshape(NW * NCH, CH)
    dst2 = ei[1].reshape(NW * NCH, CH)

    dg0, dg1 = _make_sc_deg()(dst2)
    h1 = _tc_mm1(x, W1)
    hp1 = _tc_scale(h1, dg0, dg1)
    p10, p11 = _sc_agg(64)(hp1, src2, dst2)

    b1r = b1.reshape(1, 64)
    hp2 = _tc_mid(p10, p11, hp1, dg0, dg1, b1r, W2)
    p20, p21 = _sc_agg(48)(hp2, src2, dst2)

    b2r = b2.reshape(1, 40)
    return _tc_fin(p20, p21, hp2, dg0, dg1, b2r)


# speedup vs baseline: 1.0524x; 1.0524x over previous
import functools

import jax
import jax.numpy as jnp
from jax import lax
from jax.experimental import pallas as pl
from jax.experimental.pallas import tpu as pltpu
from jax.experimental.pallas import tpu_sc as plsc

N_NODES = 10000
N_PAD = 10240
N_EDGES = 320000
NW = 32
EPW = N_EDGES // NW
CH = 125
NCH = EPW // CH
RPT = N_PAD // 16
ZCH = 128
NBUF = 5


@functools.lru_cache(maxsize=None)
def _sc_agg(D):
    mesh = plsc.VectorSubcoreMesh(core_axis_name="c", subcore_axis_name="s")

    @functools.partial(
        pl.kernel,
        mesh=mesh,
        out_type=[
            jax.ShapeDtypeStruct((N_PAD, 128), jnp.float32),
            jax.ShapeDtypeStruct((N_PAD, 128), jnp.float32),
        ],
        scratch_types=[
            pltpu.VMEM((NCH, CH), jnp.int32),
            pltpu.VMEM((NCH, CH), jnp.int32),
            [pltpu.VMEM((CH, D), jnp.float32) for _ in range(NBUF)],
            pltpu.VMEM((ZCH, D), jnp.float32),
            pltpu.VMEM_SHARED((N_PAD, D), jnp.float32),
            [pltpu.SemaphoreType.DMA for _ in range(NBUF)],
            [pltpu.SemaphoreType.DMA for _ in range(NBUF)],
            pltpu.SemaphoreType.DMA,
        ],
        compiler_params=pltpu.CompilerParams(use_tc_tiling_on_sc=False),
    )
    def k(table, ei, out0, out1, src_v, dst_v, bufs, zbuf, acc, gs, ss, zs):
        cid = lax.axis_index("c")
        sid = lax.axis_index("s")
        wid = sid * 2 + cid
        cps = pltpu.async_copy(ei.at[0, pl.ds(wid * NCH, NCH)], src_v, gs[0])
        cpd = pltpu.async_copy(ei.at[1, pl.ds(wid * NCH, NCH)], dst_v, gs[1])

        zeros = jnp.zeros((16,), jnp.float32)

        def zrow(i, _):
            for d in range(D // 16):
                zbuf[i, pl.ds(d * 16, 16)] = zeros
            return 0

        lax.fori_loop(0, ZCH, zrow, 0)
        row0 = sid * RPT
        for kk in range(RPT // ZCH):
            pltpu.async_copy(zbuf, acc.at[pl.ds(row0 + kk * ZCH, ZCH)], zs)
        for kk in range(RPT // ZCH):
            pltpu.make_async_copy(
                zbuf, acc.at[pl.ds(row0 + kk * ZCH, ZCH)], zs).wait()
        cps.wait()
        cpd.wait()
        plsc.subcore_barrier()

        def gather(j, b):
            return pltpu.async_copy(table.at[src_v.at[j]], bufs[b], gs[b])

        def scat(j, b):
            return pltpu.async_copy(bufs[b], acc.at[dst_v.at[j]], ss[b],
                                    add=True)

        for b in range(NBUF - 1):
            gather(b, b)

        def body(g, _):
            j0 = NBUF * g
            for b in range(NBUF):
                j = j0 + b
                pltpu.make_async_copy(table.at[src_v.at[j]], bufs[b],
                                      gs[b]).wait()
                scat(j, b)
                bn = (b + NBUF - 1) % NBUF

                @pl.when(j + NBUF - 1 < NCH)
                def _():
                    @pl.when(j >= 1)
                    def _():
                        pltpu.make_async_copy(
                            bufs[bn], acc.at[dst_v.at[j - 1]], ss[bn]).wait()

                    gather(j + NBUF - 1, bn)
            return 0

        lax.fori_loop(0, NCH // NBUF, body, 0)
        for jj in range(NCH - NBUF, NCH):
            pltpu.make_async_copy(
                bufs[jj % NBUF], acc.at[dst_v.at[jj]], ss[jj % NBUF]).wait()
        plsc.subcore_barrier()

        @pl.when(cid == 0)
        def _():
            pltpu.sync_copy(acc.at[pl.ds(row0, RPT)],
                            out0.at[pl.ds(row0, RPT), pl.ds(0, D)])

        @pl.when(cid == 1)
        def _():
            pltpu.sync_copy(acc.at[pl.ds(row0, RPT)],
                            out1.at[pl.ds(row0, RPT), pl.ds(0, D)])

    return k


@functools.lru_cache(maxsize=None)
def _make_sc_deg():
    mesh = plsc.VectorSubcoreMesh(core_axis_name="c", subcore_axis_name="s")
    D = 16

    @functools.partial(
        pl.kernel,
        mesh=mesh,
        out_type=[
            jax.ShapeDtypeStruct((N_PAD, 128), jnp.float32),
            jax.ShapeDtypeStruct((N_PAD, 128), jnp.float32),
        ],
        scratch_types=[
            pltpu.VMEM((NCH, CH), jnp.int32),
            pltpu.VMEM((CH, D), jnp.float32),
            pltpu.VMEM((ZCH, D), jnp.float32),
            pltpu.VMEM_SHARED((N_PAD, D), jnp.float32),
            [pltpu.SemaphoreType.DMA for _ in range(4)],
            pltpu.SemaphoreType.DMA,
            pltpu.SemaphoreType.DMA,
        ],
        compiler_params=pltpu.CompilerParams(use_tc_tiling_on_sc=False),
    )
    def k(ei, out0, out1, dst_v, buf, zbuf, acc, ss, s0, zs):
        cid = lax.axis_index("c")
        sid = lax.axis_index("s")
        wid = sid * 2 + cid
        cpd = pltpu.async_copy(ei.at[1, pl.ds(wid * NCH, NCH)], dst_v, zs)

        zeros = jnp.zeros((16,), jnp.float32)

        def zrow(i, _):
            zbuf[i, :] = zeros
            return 0

        lax.fori_loop(0, ZCH, zrow, 0)
        row0 = sid * RPT
        for kk in range(RPT // ZCH):
            pltpu.async_copy(zbuf, acc.at[pl.ds(row0 + kk * ZCH, ZCH)], s0)
        ones = jnp.ones((16,), jnp.float32)

        def orow(i, _):
            buf[i, :] = ones
            return 0

        lax.fori_loop(0, CH, orow, 0)
        for kk in range(RPT // ZCH):
            pltpu.make_async_copy(
                zbuf, acc.at[pl.ds(row0 + kk * ZCH, ZCH)], s0).wait()
        cpd.wait()
        plsc.subcore_barrier()

        def body(g, _):
            j0 = 4 * g
            for b in range(4):
                j = j0 + b

                @pl.when(j >= 4)
                def _():
                    pltpu.make_async_copy(
                        buf, acc.at[dst_v.at[j - 4]], ss[b]).wait()

                pltpu.async_copy(buf, acc.at[dst_v.at[j]], ss[b], add=True)
            return 0

        lax.fori_loop(0, NCH // 4, body, 0)
        for jj in range(NCH - 4, NCH):
            pltpu.make_async_copy(buf, acc.at[dst_v.at[jj]], ss[jj % 4]).wait()
        plsc.subcore_barrier()

        @pl.when(cid == 0)
        def _():
            pltpu.sync_copy(acc.at[pl.ds(row0, RPT)],
                            out0.at[pl.ds(row0, RPT), pl.ds(0, D)])

        @pl.when(cid == 1)
        def _():
            pltpu.sync_copy(acc.at[pl.ds(row0, RPT)],
                            out1.at[pl.ds(row0, RPT), pl.ds(0, D)])

    return k


_R = 2000


def _dinv_of(d0_ref, d1_ref):
    deg = d0_ref[:, 0:1] + d1_ref[:, 0:1] + 1.0
    return lax.rsqrt(deg)


def _mm1_body(x_ref, w_ref, o_ref):
    o_ref[...] = jnp.dot(x_ref[...], w_ref[...],
                         preferred_element_type=jnp.float32)


def _tc_mm1(x, W1):
    return pl.pallas_call(
        _mm1_body,
        grid=(N_NODES // _R,),
        in_specs=[
            pl.BlockSpec((_R, 128), lambda i: (i, 0)),
            pl.BlockSpec((128, 64), lambda i: (0, 0)),
        ],
        out_specs=pl.BlockSpec((_R, 64), lambda i: (i, 0)),
        out_shape=jax.ShapeDtypeStruct((N_NODES, 64), jnp.float32),
    )(x, W1)


def _scale_body(h_ref, d0_ref, d1_ref, o_ref):
    o_ref[...] = h_ref[...] * _dinv_of(d0_ref, d1_ref)


def _tc_scale(h, dg0, dg1):
    return pl.pallas_call(
        _scale_body,
        grid=(N_NODES // _R,),
        in_specs=[
            pl.BlockSpec((_R, 64), lambda i: (i, 0)),
            pl.BlockSpec((_R, 128), lambda i: (i, 0)),
            pl.BlockSpec((_R, 128), lambda i: (i, 0)),
        ],
        out_specs=pl.BlockSpec((_R, 64), lambda i: (i, 0)),
        out_shape=jax.ShapeDtypeStruct((N_NODES, 64), jnp.float32),
    )(h, dg0, dg1)


def _mid_body(p0_ref, p1_ref, hp_ref, d0_ref, d1_ref, b1_ref, w2_ref, o_ref):
    dinv = _dinv_of(d0_ref, d1_ref)
    agg = p0_ref[:, :64] + p1_ref[:, :64]
    t = b1_ref[...] + dinv * (agg + hp_ref[...])
    t = jnp.maximum(t, 0.0)
    w2p = jnp.pad(w2_ref[...], ((0, 0), (0, 8)))
    o_ref[...] = jnp.dot(t, w2p, preferred_element_type=jnp.float32) * dinv


def _tc_mid(p0, p1, hp, dg0, dg1, b1r, W2r):
    return pl.pallas_call(
        _mid_body,
        grid=(N_NODES // _R,),
        in_specs=[
            pl.BlockSpec((_R, 128), lambda i: (i, 0)),
            pl.BlockSpec((_R, 128), lambda i: (i, 0)),
            pl.BlockSpec((_R, 64), lambda i: (i, 0)),
            pl.BlockSpec((_R, 128), lambda i: (i, 0)),
            pl.BlockSpec((_R, 128), lambda i: (i, 0)),
            pl.BlockSpec((1, 64), lambda i: (0, 0)),
            pl.BlockSpec((64, 40), lambda i: (0, 0)),
        ],
        out_specs=pl.BlockSpec((_R, 48), lambda i: (i, 0)),
        out_shape=jax.ShapeDtypeStruct((N_NODES, 48), jnp.float32),
    )(p0, p1, hp, dg0, dg1, b1r, W2r)


def _fin_body(p0_ref, p1_ref, hp_ref, d0_ref, d1_ref, b2_ref, o_ref):
    dinv = _dinv_of(d0_ref, d1_ref)
    b2p = jnp.pad(b2_ref[...], ((0, 0), (0, 8)))
    agg = p0_ref[:, :48] + p1_ref[:, :48]
    z = b2p + dinv * (agg + hp_ref[...])
    z = jnp.maximum(z, 0.0)
    col = lax.broadcasted_iota(jnp.int32, z.shape, 1)
    valid = col < 40
    zm = jnp.where(valid, z, -jnp.inf)
    m = jnp.max(zm, axis=1, keepdims=True)
    e = jnp.where(valid, jnp.exp(z - m), 0.0)
    s = jnp.log(jnp.sum(e, axis=1, keepdims=True))
    o_ref[...] = (z - m - s)[:, :40]


def _tc_fin(p0, p1, hp, dg0, dg1, b2r):
    return pl.pallas_call(
        _fin_body,
        grid=(N_NODES // _R,),
        in_specs=[
            pl.BlockSpec((_R, 128), lambda i: (i, 0)),
            pl.BlockSpec((_R, 128), lambda i: (i, 0)),
            pl.BlockSpec((_R, 48), lambda i: (i, 0)),
            pl.BlockSpec((_R, 128), lambda i: (i, 0)),
            pl.BlockSpec((_R, 128), lambda i: (i, 0)),
            pl.BlockSpec((1, 40), lambda i: (0, 0)),
        ],
        out_specs=pl.BlockSpec((_R, 40), lambda i: (i, 0)),
        out_shape=jax.ShapeDtypeStruct((N_NODES, 40), jnp.float32),
    )(p0, p1, hp, dg0, dg1, b2r)


def kernel(x, W1, b1, W2, b2, edge_index):
    ei3 = edge_index.astype(jnp.int32).reshape(2, NW * NCH, CH)

    dg0, dg1 = _make_sc_deg()(ei3)
    h1 = _tc_mm1(x, W1)
    hp1 = _tc_scale(h1, dg0, dg1)
    p10, p11 = _sc_agg(64)(hp1, ei3)

    b1r = b1.reshape(1, 64)
    hp2 = _tc_mid(p10, p11, hp1, dg0, dg1, b1r, W2)
    p20, p21 = _sc_agg(48)(hp2, ei3)

    b2r = b2.reshape(1, 40)
    return _tc_fin(p20, p21, hp2, dg0, dg1, b2r)
